# baseline (speedup 1.0000x reference)
import functools

import jax
import jax.numpy as jnp
from jax import lax
from jax.experimental import pallas as pl
from jax.experimental.pallas import tpu as pltpu
from jax.experimental.pallas import tpu_sc as plsc

_N = 10000
_D = 256
_E = 160000
_Q = _D // 4
_NTILES = 16
_EDGES_PER_TILE = _E // _NTILES
_CHUNK = 400
_NCHUNK = _EDGES_PER_TILE // _CHUNK
_NSLOT = 2
_NPASS = 2
_WB_TILES = 10
_WB_ROWS = _N // _WB_TILES


def _make_sc_agg():
    mesh = plsc.VectorSubcoreMesh(core_axis_name="c", subcore_axis_name="s")

    scratch = (
        [pltpu.VMEM((_EDGES_PER_TILE,), jnp.int32),
         pltpu.VMEM((_EDGES_PER_TILE,), jnp.int32)]
        + [pltpu.VMEM((_CHUNK, _Q), jnp.float32) for _ in range(_NSLOT)]
        + [pltpu.VMEM_SHARED((_N, _Q), jnp.float32)]
        + [pltpu.SemaphoreType.DMA for _ in range(2 * _NSLOT)]
    )

    @functools.partial(
        pl.kernel,
        mesh=mesh,
        out_type=jax.ShapeDtypeStruct((_NPASS, 2, _N, _Q), jnp.float32),
        scratch_types=scratch,
        compiler_params=pltpu.CompilerParams(use_tc_tiling_on_sc=False),
    )
    def sc_agg(h4_hbm, idx4_hbm, dst_hbm, zeros_hbm, out_hbm,
               idx_big, dst_big, *rest):
        rows = rest[:_NSLOT]
        acc_sh = rest[_NSLOT]
        gsem = rest[_NSLOT + 1:2 * _NSLOT + 1]
        ssem = rest[2 * _NSLOT + 1:]
        c = lax.axis_index("c")
        s = lax.axis_index("s")
        row0 = s * _WB_ROWS

        pltpu.sync_copy(dst_hbm.at[pl.ds(s * _EDGES_PER_TILE, _EDGES_PER_TILE)],
                        dst_big)

        def islice(ref, k):
            return ref.at[pl.ds(k * _CHUNK, _CHUNK)]

        def gather(k, rows, sem):
            pltpu.async_copy(h4_hbm.at[islice(idx_big, k)], rows, sem)

        def drain_gather(k, rows, sem):
            pltpu.make_async_copy(h4_hbm.at[islice(idx_big, k)], rows, sem).wait()

        def scatter(k, rows, sem):
            pltpu.async_copy(rows, acc_sh.at[islice(dst_big, k)], sem, add=True)

        def drain_scatter(k, rows, sem):
            pltpu.make_async_copy(rows, acc_sh.at[islice(dst_big, k)], sem).wait()

        for p in range(_NPASS):
            qbase = (2 * p + c) * _E + s * _EDGES_PER_TILE
            pltpu.sync_copy(idx4_hbm.at[pl.ds(qbase, _EDGES_PER_TILE)], idx_big)

            pltpu.async_copy(h4_hbm.at[islice(idx_big, 0)], rows[0], gsem[0])

            @pl.when(s < _WB_TILES)
            def _zero():
                pltpu.sync_copy(zeros_hbm, acc_sh.at[pl.ds(row0, _WB_ROWS)])

            plsc.subcore_barrier()

            for k in range(1, _NSLOT):
                gather(k, rows[k % _NSLOT], gsem[k % _NSLOT])
                prev = (k - 1) % _NSLOT
                drain_gather(k - 1, rows[prev], gsem[prev])
                scatter(k - 1, rows[prev], ssem[prev])

            steady_end = ((_NCHUNK - 1) // _NSLOT) * _NSLOT

            def ring_body(j, carry):
                for r in range(_NSLOT):
                    k = _NSLOT * j + r
                    prev = (r + _NSLOT - 1) % _NSLOT
                    drain_scatter(k - _NSLOT, rows[r], ssem[r])
                    gather(k, rows[r], gsem[r])
                    drain_gather(k - 1, rows[prev], gsem[prev])
                    scatter(k - 1, rows[prev], ssem[prev])
                return carry

            lax.fori_loop(1, steady_end // _NSLOT, ring_body, 0)

            for k in range(steady_end, _NCHUNK + _NSLOT):
                r = k % _NSLOT
                prev = (r + _NSLOT - 1) % _NSLOT
                if 0 <= k - _NSLOT <= _NCHUNK - 1:
                    drain_scatter(k - _NSLOT, rows[r], ssem[r])
                if k < _NCHUNK:
                    gather(k, rows[r], gsem[r])
                if k - 1 <= _NCHUNK - 1:
                    drain_gather(k - 1, rows[prev], gsem[prev])
                    scatter(k - 1, rows[prev], ssem[prev])

            plsc.subcore_barrier()

            @pl.when(s < _WB_TILES)
            def _writeback():
                pltpu.sync_copy(acc_sh.at[pl.ds(row0, _WB_ROWS)],
                                out_hbm.at[p, c, pl.ds(row0, _WB_ROWS)])

            if p + 1 < _NPASS:
                plsc.subcore_barrier()

    return sc_agg


_sc_agg = _make_sc_agg()


def _dense_body(h_ref, a0_ref, a1_ref, a2_ref, a3_ref,
                w1_ref, b1_ref, w2_ref, b2_ref,
                sc_ref, gamma_ref, beta_ref, out_ref):
    h = h_ref[...]
    agg = jnp.concatenate(
        [a0_ref[0, 0], a1_ref[0, 0], a2_ref[0, 0], a3_ref[0, 0]], axis=1)
    z = h * sc_ref[...] + agg
    t = jnp.maximum(jnp.dot(z, w1_ref[...], preferred_element_type=jnp.float32)
                    + b1_ref[...], 0.0)
    r = jnp.dot(t, w2_ref[...], preferred_element_type=jnp.float32) \
        + b2_ref[...] + h
    mu = jnp.mean(r, axis=1, keepdims=True)
    d = r - mu
    var = jnp.mean(d * d, axis=1, keepdims=True)
    ln = d * lax.rsqrt(var + 1e-5) * gamma_ref[...] + beta_ref[...]
    out_ref[...] = jnp.maximum(ln, 0.0)


_BLK = 1000


def _dense(h, agg4, W1, b1, W2, b2, scale, gamma, beta):
    nblk = _N // _BLK
    full = lambda i: (0, 0)

    def qspec(p, c):
        return pl.BlockSpec((1, 1, _BLK, _Q), lambda i, p=p, c=c: (p, c, i, 0))

    return pl.pallas_call(
        _dense_body,
        grid=(nblk,),
        in_specs=[
            pl.BlockSpec((_BLK, _D), lambda i: (i, 0)),
            qspec(0, 0),
            qspec(0, 1),
            qspec(1, 0),
            qspec(1, 1),
            pl.BlockSpec((_D, _D), full),
            pl.BlockSpec((1, _D), full),
            pl.BlockSpec((_D, _D), full),
            pl.BlockSpec((1, _D), full),
            pl.BlockSpec((1, 1), full),
            pl.BlockSpec((1, _D), full),
            pl.BlockSpec((1, _D), full),
        ],
        out_specs=pl.BlockSpec((_BLK, _D), lambda i: (i, 0)),
        out_shape=jax.ShapeDtypeStruct((_N, _D), jnp.float32),
    )(h, agg4, agg4, agg4, agg4, W1, b1, W2, b2, scale, gamma, beta)


def kernel(h, edge_index, W1, b1, W2, b2, eps, gamma, beta):
    src = edge_index[0]
    dst = edge_index[1]
    base = src * 4
    idx4 = jnp.concatenate([base, base + 1, base + 2, base + 3])
    h4 = h.reshape(4 * _N, _Q)
    zeros = jnp.zeros((_WB_ROWS, _Q), jnp.float32)
    agg4 = _sc_agg(h4, idx4, dst, zeros)
    scale = jnp.reshape(1.0 + eps, (1, 1))
    return _dense(h, agg4,
                  W1, b1.reshape(1, _D), W2, b2.reshape(1, _D),
                  scale, gamma.reshape(1, _D), beta.reshape(1, _D))


# speedup vs baseline: 1.1398x; 1.1398x over previous
import functools

import jax
import jax.numpy as jnp
from jax import lax
from jax.experimental import pallas as pl
from jax.experimental.pallas import tpu as pltpu
from jax.experimental.pallas import tpu_sc as plsc

_N = 10000
_D = 256
_E = 160000
_HALF = _D // 2
_NTILES = 16
_EDGES_PER_TILE = _E // _NTILES
_CHUNK = 80
_NCHUNK = _EDGES_PER_TILE // _CHUNK
_NSLOT = 3
_WB_TILES = 10
_WB_ROWS = _N // _WB_TILES


def _make_sc_agg():
    mesh = plsc.VectorSubcoreMesh(core_axis_name="c", subcore_axis_name="s")

    scratch = (
        [pltpu.VMEM((_EDGES_PER_TILE,), jnp.int32),
         pltpu.VMEM((_EDGES_PER_TILE,), jnp.int32)]
        + [pltpu.VMEM((_CHUNK, _HALF), jnp.float32) for _ in range(_NSLOT)]
        + [pltpu.VMEM_SHARED((_N, _HALF), jnp.float32)]
        + [pltpu.SemaphoreType.DMA for _ in range(2 * _NSLOT)]
    )

    @functools.partial(
        pl.kernel,
        mesh=mesh,
        out_type=jax.ShapeDtypeStruct((2, _N, _HALF), jnp.float32),
        scratch_types=scratch,
    )
    def sc_agg(h2_hbm, idx2_hbm, dst_hbm, zeros_hbm, out_hbm,
               idx_big, dst_big, *rest):
        rows = rest[:_NSLOT]
        acc_sh = rest[_NSLOT]
        gsem = rest[_NSLOT + 1:2 * _NSLOT + 1]
        ssem = rest[2 * _NSLOT + 1:]
        c = lax.axis_index("c")
        s = lax.axis_index("s")

        pltpu.sync_copy(
            idx2_hbm.at[pl.ds(c * _E + s * _EDGES_PER_TILE, _EDGES_PER_TILE)],
            idx_big)
        pltpu.sync_copy(dst_hbm.at[pl.ds(s * _EDGES_PER_TILE, _EDGES_PER_TILE)],
                        dst_big)

        def islice(ref, k):
            return ref.at[pl.ds(k * _CHUNK, _CHUNK)]

        pltpu.async_copy(h2_hbm.at[islice(idx_big, 0)], rows[0], gsem[0])

        row0 = s * _WB_ROWS

        @pl.when(s < _WB_TILES)
        def _zero():
            pltpu.sync_copy(zeros_hbm, acc_sh.at[pl.ds(row0, _WB_ROWS)])

        plsc.subcore_barrier()

        def gather(k, rows, sem):
            pltpu.async_copy(h2_hbm.at[islice(idx_big, k)], rows, sem)

        def drain_gather(k, rows, sem):
            pltpu.make_async_copy(h2_hbm.at[islice(idx_big, k)], rows, sem).wait()

        def scatter(k, rows, sem):
            pltpu.async_copy(rows, acc_sh.at[islice(dst_big, k)], sem, add=True)

        def drain_scatter(k, rows, sem):
            pltpu.make_async_copy(rows, acc_sh.at[islice(dst_big, k)], sem).wait()

        def step(k, r, first=None, last=None):
            prev = (r + _NSLOT - 1) % _NSLOT
            if first is None or k >= _NSLOT:
                drain_scatter(k - _NSLOT, rows[r], ssem[r])
            if first is None or (k >= 1 and (last is None or k < last)):
                gather(k, rows[r], gsem[r])
            if first is None or k >= 1:
                drain_gather(k - 1, rows[prev], gsem[prev])
                scatter(k - 1, rows[prev], ssem[prev])

        for k in range(1, _NSLOT):
            step(k, k % _NSLOT, first=0, last=_NCHUNK)

        steady_end = ((_NCHUNK - 1) // _NSLOT) * _NSLOT

        def ring_body(j, carry):
            for r in range(_NSLOT):
                step(_NSLOT * j + r, r)
            return carry

        lax.fori_loop(1, steady_end // _NSLOT, ring_body, 0)

        for k in range(steady_end, _NCHUNK + _NSLOT):
            r = k % _NSLOT
            prev = (r + _NSLOT - 1) % _NSLOT
            if k - _NSLOT >= 0 and k - _NSLOT <= _NCHUNK - 1:
                drain_scatter(k - _NSLOT, rows[r], ssem[r])
            if k < _NCHUNK:
                gather(k, rows[r], gsem[r])
            if k - 1 <= _NCHUNK - 1:
                drain_gather(k - 1, rows[prev], gsem[prev])
                scatter(k - 1, rows[prev], ssem[prev])

        plsc.subcore_barrier()

        @pl.when(s < _WB_TILES)
        def _writeback():
            pltpu.sync_copy(acc_sh.at[pl.ds(row0, _WB_ROWS)],
                            out_hbm.at[c, pl.ds(row0, _WB_ROWS)])

    return sc_agg


_sc_agg = _make_sc_agg()


def _dense_body(h_ref, a0_ref, a1_ref, w1_ref, b1_ref, w2_ref, b2_ref,
                sc_ref, gamma_ref, beta_ref, out_ref):
    h = h_ref[...]
    agg = jnp.concatenate([a0_ref[0], a1_ref[0]], axis=1)
    z = h * sc_ref[...] + agg
    t = jnp.maximum(jnp.dot(z, w1_ref[...], preferred_element_type=jnp.float32)
                    + b1_ref[...], 0.0)
    r = jnp.dot(t, w2_ref[...], preferred_element_type=jnp.float32) \
        + b2_ref[...] + h
    mu = jnp.mean(r, axis=1, keepdims=True)
    d = r - mu
    var = jnp.mean(d * d, axis=1, keepdims=True)
    ln = d * lax.rsqrt(var + 1e-5) * gamma_ref[...] + beta_ref[...]
    out_ref[...] = jnp.maximum(ln, 0.0)


_BLK = 2000


def _dense(h, agg2, W1, b1, W2, b2, scale, gamma, beta):
    nblk = _N // _BLK
    full = lambda i: (0, 0)
    return pl.pallas_call(
        _dense_body,
        grid=(nblk,),
        in_specs=[
            pl.BlockSpec((_BLK, _D), lambda i: (i, 0)),
            pl.BlockSpec((1, _BLK, _HALF), lambda i: (0, i, 0)),
            pl.BlockSpec((1, _BLK, _HALF), lambda i: (1, i, 0)),
            pl.BlockSpec((_D, _D), full),
            pl.BlockSpec((1, _D), full),
            pl.BlockSpec((_D, _D), full),
            pl.BlockSpec((1, _D), full),
            pl.BlockSpec((1, 1), full),
            pl.BlockSpec((1, _D), full),
            pl.BlockSpec((1, _D), full),
        ],
        out_specs=pl.BlockSpec((_BLK, _D), lambda i: (i, 0)),
        out_shape=jax.ShapeDtypeStruct((_N, _D), jnp.float32),
    )(h, agg2, agg2, W1, b1, W2, b2, scale, gamma, beta)


def kernel(h, edge_index, W1, b1, W2, b2, eps, gamma, beta):
    src = edge_index[0]
    dst = edge_index[1]
    idx2 = jnp.concatenate([src * 2, src * 2 + 1])
    h2 = h.reshape(2 * _N, _HALF)
    zeros = jnp.zeros((_WB_ROWS, _HALF), jnp.float32)
    agg2 = _sc_agg(h2, idx2, dst, zeros)
    scale = jnp.reshape(1.0 + eps, (1, 1))
    return _dense(h, agg2,
                  W1, b1.reshape(1, _D), W2, b2.reshape(1, _D),
                  scale, gamma.reshape(1, _D), beta.reshape(1, _D))


# speedup vs baseline: 1.1400x; 1.0002x over previous
import functools

import jax
import jax.numpy as jnp
from jax import lax
from jax.experimental import pallas as pl
from jax.experimental.pallas import tpu as pltpu
from jax.experimental.pallas import tpu_sc as plsc

_N = 10000
_D = 256
_E = 160000
_HALF = _D // 2
_NTILES = 16
_EDGES_PER_TILE = _E // _NTILES
_CHUNK = 80
_NCHUNK = _EDGES_PER_TILE // _CHUNK
_NSLOT = 3
_WB_TILES = 10
_WB_ROWS = _N // _WB_TILES


def _make_sc_agg():
    mesh = plsc.VectorSubcoreMesh(core_axis_name="c", subcore_axis_name="s")

    scratch = (
        [pltpu.VMEM((_EDGES_PER_TILE,), jnp.int32),
         pltpu.VMEM((_EDGES_PER_TILE,), jnp.int32)]
        + [pltpu.VMEM((_CHUNK, _HALF), jnp.float32) for _ in range(_NSLOT)]
        + [pltpu.VMEM_SHARED((_N, _HALF), jnp.float32)]
        + [pltpu.SemaphoreType.DMA for _ in range(2 * _NSLOT)]
    )

    @functools.partial(
        pl.kernel,
        mesh=mesh,
        out_type=jax.ShapeDtypeStruct((2, _N, _HALF), jnp.float32),
        scratch_types=scratch,
    )
    def sc_agg(h2_hbm, idx2_hbm, dst_hbm, zeros_hbm, out_hbm,
               idx_big, dst_big, *rest):
        rows = rest[:_NSLOT]
        acc_sh = rest[_NSLOT]
        gsem = rest[_NSLOT + 1:2 * _NSLOT + 1]
        ssem = rest[2 * _NSLOT + 1:]
        c = lax.axis_index("c")
        s = lax.axis_index("s")

        pltpu.sync_copy(
            idx2_hbm.at[pl.ds(c * _E + s * _EDGES_PER_TILE, _EDGES_PER_TILE)],
            idx_big)
        pltpu.sync_copy(dst_hbm.at[pl.ds(s * _EDGES_PER_TILE, _EDGES_PER_TILE)],
                        dst_big)

        def islice(ref, k):
            return ref.at[pl.ds(k * _CHUNK, _CHUNK)]

        pltpu.async_copy(h2_hbm.at[islice(idx_big, 0)], rows[0], gsem[0])

        row0 = s * _WB_ROWS

        @pl.when(s < _WB_TILES)
        def _zero():
            pltpu.sync_copy(zeros_hbm, acc_sh.at[pl.ds(row0, _WB_ROWS)])

        plsc.subcore_barrier()

        def gather(k, rows, sem):
            pltpu.async_copy(h2_hbm.at[islice(idx_big, k)], rows, sem)

        def drain_gather(k, rows, sem):
            pltpu.make_async_copy(h2_hbm.at[islice(idx_big, k)], rows, sem).wait()

        def scatter(k, rows, sem):
            pltpu.async_copy(rows, acc_sh.at[islice(dst_big, k)], sem, add=True)

        def drain_scatter(k, rows, sem):
            pltpu.make_async_copy(rows, acc_sh.at[islice(dst_big, k)], sem).wait()

        def step(k, r, first=None, last=None):
            prev = (r + _NSLOT - 1) % _NSLOT
            if first is None or k >= _NSLOT:
                drain_scatter(k - _NSLOT, rows[r], ssem[r])
            if first is None or (k >= 1 and (last is None or k < last)):
                gather(k, rows[r], gsem[r])
            if first is None or k >= 1:
                drain_gather(k - 1, rows[prev], gsem[prev])
                scatter(k - 1, rows[prev], ssem[prev])

        for k in range(1, _NSLOT):
            step(k, k % _NSLOT, first=0, last=_NCHUNK)

        steady_end = ((_NCHUNK - 1) // _NSLOT) * _NSLOT

        def ring_body(j, carry):
            for r in range(_NSLOT):
                step(_NSLOT * j + r, r)
            return carry

        lax.fori_loop(1, steady_end // _NSLOT, ring_body, 0)

        for k in range(steady_end, _NCHUNK + _NSLOT):
            r = k % _NSLOT
            prev = (r + _NSLOT - 1) % _NSLOT
            if k - _NSLOT >= 0 and k - _NSLOT <= _NCHUNK - 1:
                drain_scatter(k - _NSLOT, rows[r], ssem[r])
            if k < _NCHUNK:
                gather(k, rows[r], gsem[r])
            if k - 1 <= _NCHUNK - 1:
                drain_gather(k - 1, rows[prev], gsem[prev])
                scatter(k - 1, rows[prev], ssem[prev])

        plsc.subcore_barrier()

        @pl.when(s < _WB_TILES)
        def _writeback():
            pltpu.sync_copy(acc_sh.at[pl.ds(row0, _WB_ROWS)],
                            out_hbm.at[c, pl.ds(row0, _WB_ROWS)])

    return sc_agg


_sc_agg = _make_sc_agg()


def _dense_body(h_ref, a0_ref, a1_ref, w1_ref, b1_ref, w2_ref, b2_ref,
                sc_ref, gamma_ref, beta_ref, out_ref):
    h = h_ref[...]
    agg = jnp.concatenate([a0_ref[0], a1_ref[0]], axis=1)
    z = h * sc_ref[...] + agg
    bf = jnp.bfloat16
    t = jnp.maximum(
        jnp.dot(z.astype(bf), w1_ref[...].astype(bf),
                preferred_element_type=jnp.float32) + b1_ref[...], 0.0)
    r = jnp.dot(t.astype(bf), w2_ref[...].astype(bf),
                preferred_element_type=jnp.float32) + b2_ref[...] + h
    mu = jnp.mean(r, axis=1, keepdims=True)
    d = r - mu
    var = jnp.mean(d * d, axis=1, keepdims=True)
    ln = d * lax.rsqrt(var + 1e-5) * gamma_ref[...] + beta_ref[...]
    out_ref[...] = jnp.maximum(ln, 0.0)


_BLK = 2000


def _dense(h, agg2, W1, b1, W2, b2, scale, gamma, beta):
    nblk = _N // _BLK
    full = lambda i: (0, 0)
    return pl.pallas_call(
        _dense_body,
        grid=(nblk,),
        in_specs=[
            pl.BlockSpec((_BLK, _D), lambda i: (i, 0)),
            pl.BlockSpec((1, _BLK, _HALF), lambda i: (0, i, 0)),
            pl.BlockSpec((1, _BLK, _HALF), lambda i: (1, i, 0)),
            pl.BlockSpec((_D, _D), full),
            pl.BlockSpec((1, _D), full),
            pl.BlockSpec((_D, _D), full),
            pl.BlockSpec((1, _D), full),
            pl.BlockSpec((1, 1), full),
            pl.BlockSpec((1, _D), full),
            pl.BlockSpec((1, _D), full),
        ],
        out_specs=pl.BlockSpec((_BLK, _D), lambda i: (i, 0)),
        out_shape=jax.ShapeDtypeStruct((_N, _D), jnp.float32),
    )(h, agg2, agg2, W1, b1, W2, b2, scale, gamma, beta)


def kernel(h, edge_index, W1, b1, W2, b2, eps, gamma, beta):
    src = edge_index[0]
    dst = edge_index[1]
    idx2 = jnp.concatenate([src * 2, src * 2 + 1])
    h2 = h.reshape(2 * _N, _HALF)
    zeros = jnp.zeros((_WB_ROWS, _HALF), jnp.float32)
    agg2 = _sc_agg(h2, idx2, dst, zeros)
    scale = jnp.reshape(1.0 + eps, (1, 1))
    return _dense(h, agg2,
                  W1, b1.reshape(1, _D), W2, b2.reshape(1, _D),
                  scale, gamma.reshape(1, _D), beta.reshape(1, _D))
